# baseline (speedup 1.0000x reference)
import functools

import jax
import jax.numpy as jnp
from jax.experimental import pallas as pl
from jax.experimental.pallas import tpu as pltpu

_TOPK = 4


def _mm(a, b, ca, cb):
    return jax.lax.dot_general(
        a.astype(jnp.bfloat16), b.astype(jnp.bfloat16),
        (((ca,), (cb,)), ((), ())), preferred_element_type=jnp.float32
    )


def _kern(x_ref, Wwh_ref, hc_ref, Wqkv_ref, Wp_ref, wg_ref, W1_ref, W2_ref,
          b2_ref, b2s_ref, out_ref, *, heads, scale):
    P, T = x_ref.shape[0], x_ref.shape[1]
    E = b2_ref.shape[0]
    HS = Wqkv_ref.shape[0] // (3 * heads)

    ri = jax.lax.broadcasted_iota(jnp.int32, (T, T), 0)
    ci = jax.lax.broadcasted_iota(jnp.int32, (T, T), 1)
    causal = ci <= ri

    h2_rows = []
    for p in range(P):
        h = _mm(x_ref[p], Wwh_ref[...], 1, 1) - hc_ref[...]
        qkv = _mm(h, Wqkv_ref[...], 1, 1)
        att_heads = []
        for hh in range(heads):
            q = qkv[:, hh * HS:(hh + 1) * HS]
            k = qkv[:, (heads + hh) * HS:(heads + hh + 1) * HS]
            v = qkv[:, (2 * heads + hh) * HS:(2 * heads + hh + 1) * HS]
            wei = _mm(q, k, 1, 1) * scale
            wei = jnp.exp(jnp.where(causal, wei, -1e30))
            wei = wei * (1.0 / jnp.sum(wei, axis=1, keepdims=True))
            att_heads.append(_mm(wei, v, 1, 0))
        att_heads.append(jnp.ones((T, 1), dtype=jnp.float32))
        att = jnp.concatenate(att_heads, axis=1)
        h2_rows.append(_mm(att, Wp_ref[...], 1, 1))
    h2a = jnp.concatenate(h2_rows, axis=0)
    N = h2a.shape[0]

    m = _mm(h2a, wg_ref[...], 1, 0)
    jidx = jax.lax.broadcasted_iota(jnp.int32, (N, E), 1)
    keeps = []
    for e in range(E):
        me = m[:, e:e + 1]
        beats = ((m > me) | ((m == me) & (jidx < e))).astype(jnp.float32)
        rank = jnp.sum(beats, axis=1, keepdims=True)
        keeps.append(jnp.where(rank < _TOPK, 1.0, 0.0))
    keep = jnp.concatenate(keeps, axis=1)
    masked = jnp.where(keep > 0.5, m, -1e30)
    ex = jnp.exp(masked - jnp.max(masked, axis=1, keepdims=True))
    ex = ex * keep
    g = ex * (1.0 / jnp.sum(ex, axis=1, keepdims=True))

    h2b = h2a.astype(jnp.bfloat16)
    D = out_ref.shape[2]
    S1 = jnp.zeros((N, D), dtype=jnp.float32)
    S2 = jnp.zeros((N, D), dtype=jnp.float32)
    for e in range(E):
        t = jnp.maximum(_mm(h2b, W1_ref[e], 1, 1), 0.0)
        o = _mm(t, W2_ref[e], 1, 1)
        S2 = S2 + o
        S1 = S1 + g[:, e:e + 1] * o
    S2 = S2 + b2s_ref[...]
    S1 = S1 + _mm(g, b2_ref[...], 1, 0)
    out = S1 * S2
    out_ref[...] = out.reshape(out_ref.shape)


@jax.jit
def kernel(x, b_wh, W_wh, Wk, Wq, Wv, W_proj, b_proj, w_gate, W1, b1, W2, b2):
    B, T, D_IN = x.shape
    D = W_wh.shape[0]
    H, HS, _ = Wq.shape
    E = w_gate.shape[1]
    D2 = W1.shape[1]
    f32 = jnp.float32

    Wqkv = jnp.concatenate(
        [Wq.reshape(H * HS, D), Wk.reshape(H * HS, D), Wv.reshape(H * HS, D)],
        axis=0)
    hc = jnp.dot(W_wh, b_wh, preferred_element_type=f32).reshape(1, D)
    Wp_aug = jnp.concatenate([W_proj, b_proj[:, None]], axis=1)
    ones_row = jnp.concatenate(
        [jnp.zeros((1, H * HS), f32), jnp.ones((1, 1), f32)], axis=1)
    Wp_aug = jnp.concatenate([Wp_aug, ones_row], axis=0)
    wg_aug = jnp.concatenate([w_gate, jnp.zeros((1, E), f32)], axis=0)
    W1_aug = jnp.concatenate([W1, b1[:, :, None]], axis=2)
    b2_sum = jnp.sum(b2, axis=0).reshape(1, D)

    P = 4 if B % 4 == 0 else 1
    full = lambda shape: pl.BlockSpec(shape, lambda b: (0,) * len(shape))
    out = pl.pallas_call(
        functools.partial(_kern, heads=H, scale=D ** -0.5),
        grid=(B // P,),
        in_specs=[
            pl.BlockSpec((P, T, D_IN), lambda b: (b, 0, 0)),
            full((D, D_IN)),
            full((1, D)),
            full((3 * H * HS, D)),
            full((D + 1, H * HS + 1)),
            full((D + 1, E)),
            full((E, D2, D + 1)),
            full((E, D, D2)),
            full((E, D)),
            full((1, D)),
        ],
        out_specs=pl.BlockSpec((P, T, D), lambda b: (b, 0, 0)),
        out_shape=jax.ShapeDtypeStruct((B, T, D), f32),
        compiler_params=pltpu.CompilerParams(
            dimension_semantics=("parallel",),
        ),
    )(x, W_wh, hc, Wqkv, Wp_aug, wg_aug, W1_aug, W2, b2, b2_sum)
    return out


# speedup vs baseline: 1.1393x; 1.1393x over previous
import functools

import jax
import jax.numpy as jnp
from jax.experimental import pallas as pl
from jax.experimental.pallas import tpu as pltpu

_TOPK = 4
_BF = jnp.bfloat16
_F32 = jnp.float32


def _mm(a, b, ca, cb, out_dtype=None):
    r = jax.lax.dot_general(
        a, b, (((ca,), (cb,)), ((), ())), preferred_element_type=jnp.float32)
    return r.astype(out_dtype) if out_dtype is not None else r


def _kern(x_ref, bwh_ref, Wwh_ref, Wqkv_ref, Wp_ref, bp_ref,
          wg_ref, W1_ref, b1_ref, W2_ref, b2_ref, out_ref, *, heads, scale):
    P, T = x_ref.shape[0], x_ref.shape[1]
    E = wg_ref.shape[1]
    HS = Wqkv_ref.shape[0] // (3 * heads)

    ri = jax.lax.broadcasted_iota(jnp.int32, (T, T), 0)
    ci = jax.lax.broadcasted_iota(jnp.int32, (T, T), 1)
    causal = ci <= ri

    h2_rows = []
    for p in range(P):
        x = (x_ref[p] - bwh_ref[...]).astype(_BF)
        h = _mm(x, Wwh_ref[...], 1, 1, _BF)
        qkv = _mm(h, Wqkv_ref[...], 1, 1, _BF)
        att_heads = []
        for hh in range(heads):
            q = qkv[:, hh * HS:(hh + 1) * HS]
            k = qkv[:, (heads + hh) * HS:(heads + hh + 1) * HS]
            v = qkv[:, (2 * heads + hh) * HS:(2 * heads + hh + 1) * HS]
            wei = _mm(q, k, 1, 1) * scale
            wei = jnp.exp(jnp.where(causal, wei, -1e30))
            rs = 1.0 / jnp.sum(wei, axis=1, keepdims=True)
            av = _mm(wei.astype(_BF), v, 1, 0)
            att_heads.append((av * rs).astype(_BF))
        att = jnp.concatenate(att_heads, axis=1)
        h2_rows.append(_mm(att, Wp_ref[...], 1, 1) + bp_ref[...])
    h2 = jnp.concatenate(h2_rows, axis=0)
    h2b = h2.astype(_BF)

    m = _mm(h2b, wg_ref[...], 1, 0)
    jidx = jax.lax.broadcasted_iota(jnp.int32, (P * T, E), 1)
    keeps = []
    for e in range(E):
        me = m[:, e:e + 1]
        beats = ((m > me) | ((m == me) & (jidx < e))).astype(_F32)
        rank = jnp.sum(beats, axis=1, keepdims=True)
        keeps.append(jnp.where(rank < _TOPK, 1.0, 0.0))
    keep = jnp.concatenate(keeps, axis=1)
    masked = jnp.where(keep > 0.5, m, -1e30)
    ex = jnp.exp(masked - jnp.max(masked, axis=1, keepdims=True))
    ex = ex * keep
    g = ex * (1.0 / jnp.sum(ex, axis=1, keepdims=True))

    b1 = b1_ref[...]
    b2 = b2_ref[...]
    S1 = jnp.zeros_like(h2)
    S2 = jnp.zeros_like(h2)
    for e in range(E):
        t = jnp.maximum(_mm(h2b, W1_ref[e], 1, 1, _BF) + b1[e:e + 1, :], 0)
        o = _mm(t, W2_ref[e], 1, 1) + b2[e:e + 1, :]
        S2 = S2 + o
        S1 = S1 + g[:, e:e + 1] * o
    out = S1 * S2
    out_ref[...] = out.reshape(out_ref.shape)


@jax.jit
def kernel(x, b_wh, W_wh, Wk, Wq, Wv, W_proj, b_proj, w_gate, W1, b1, W2, b2):
    B, T, D_IN = x.shape
    D = W_wh.shape[0]
    H, HS, _ = Wq.shape
    E = w_gate.shape[1]
    D2 = W1.shape[1]

    Wqkv = jnp.concatenate(
        [Wq.reshape(H * HS, D), Wk.reshape(H * HS, D), Wv.reshape(H * HS, D)],
        axis=0)

    P = 4 if B % 4 == 0 else 1
    full = lambda shape: pl.BlockSpec(shape, lambda b: (0,) * len(shape))
    out = pl.pallas_call(
        functools.partial(_kern, heads=H, scale=D ** -0.5),
        grid=(B // P,),
        in_specs=[
            pl.BlockSpec((P, T, D_IN), lambda b: (b, 0, 0)),
            full((1, D_IN)),
            full((D, D_IN)),
            full((3 * H * HS, D)),
            full((D, H * HS)),
            full((1, D)),
            full((D, E)),
            full((E, D2, D)),
            full((E, D2)),
            full((E, D, D2)),
            full((E, D)),
        ],
        out_specs=pl.BlockSpec((P, T, D), lambda b: (b, 0, 0)),
        out_shape=jax.ShapeDtypeStruct((B, T, D), _F32),
        compiler_params=pltpu.CompilerParams(
            dimension_semantics=("parallel",),
        ),
    )(x, b_wh.reshape(1, D_IN), W_wh.astype(_BF), Wqkv.astype(_BF),
      W_proj.astype(_BF), b_proj.reshape(1, D), w_gate.astype(_BF),
      W1.astype(_BF), b1.astype(_BF), W2.astype(_BF), b2)
    return out
